# Initial kernel scaffold; baseline (speedup 1.0000x reference)
import jax
import jax.numpy as jnp
from jax.experimental import pallas as pl


def kernel(node_edge_feat, dist_feat_order, dist_feat, W_fc, b_fc, W_g, a_src_g, a_dst_g, W_s, W_e_s, a_src_s, a_dst_s, a_edge_s, srcs, dsts, nids, eids, e2n_edge_index, e2e_edge_index, nlod, elod):
    raise NotImplementedError("write your pallas kernel here")


# speedup vs baseline: 1.5062x; 1.5062x over previous
import functools

import jax
import jax.numpy as jnp
from jax.experimental import pallas as pl
from jax.experimental.pallas import tpu as pltpu

N_NODES = 10000
N_EDGES = 320000
H = 128
ROW_BLK = 1000


def _mlp_body(src_ref, dst_ref, dist_ref, w1_ref, w2_ref, w3_ref, b_ref, o_ref):
    acc = jnp.dot(src_ref[...], w1_ref[...], preferred_element_type=jnp.float32)
    acc = acc + jnp.dot(dst_ref[...], w2_ref[...], preferred_element_type=jnp.float32)
    acc = acc + jnp.dot(dist_ref[...], w3_ref[...], preferred_element_type=jnp.float32)
    o_ref[...] = jnp.maximum(acc + b_ref[...], 0.0)


def _edge_mlp(src_feat, dst_feat, dist_feat, w1, w2, w3, b):
    n = src_feat.shape[0]
    grid = n // ROW_BLK
    row_spec = pl.BlockSpec((ROW_BLK, H), lambda i: (i, 0))
    w_spec = pl.BlockSpec((H, H), lambda i: (0, 0))
    b_spec = pl.BlockSpec((1, H), lambda i: (0, 0))
    return pl.pallas_call(
        _mlp_body,
        grid=(grid,),
        in_specs=[row_spec, row_spec, row_spec, w_spec, w_spec, w_spec, b_spec],
        out_specs=row_spec,
        out_shape=jax.ShapeDtypeStruct((n, H), jnp.float32),
    )(src_feat, dst_feat, dist_feat, w1, w2, w3, b.reshape(1, H))


def _mm_relu_body(x_ref, w_ref, o_ref):
    o_ref[...] = jnp.maximum(
        jnp.dot(x_ref[...], w_ref[...], preferred_element_type=jnp.float32), 0.0)


def _mm_relu(x, w):
    n = x.shape[0]
    grid = n // ROW_BLK
    row_spec = pl.BlockSpec((ROW_BLK, H), lambda i: (i, 0))
    w_spec = pl.BlockSpec((H, H), lambda i: (0, 0))
    return pl.pallas_call(
        _mm_relu_body,
        grid=(grid,),
        in_specs=[row_spec, w_spec],
        out_specs=row_spec,
        out_shape=jax.ShapeDtypeStruct((n, H), jnp.float32),
    )(x, w)


def kernel(node_edge_feat, dist_feat_order, dist_feat, W_fc, b_fc, W_g, a_src_g,
           a_dst_g, W_s, W_e_s, a_src_s, a_dst_s, a_edge_s, srcs, dsts, nids,
           eids, e2n_edge_index, e2e_edge_index, nlod, elod):
    w1, w2, w3 = W_fc[:H], W_fc[H:2 * H], W_fc[2 * H:]
    node_feat = node_edge_feat[:N_NODES]

    src_feat = jnp.take(node_edge_feat, srcs, axis=0)
    dst_feat = jnp.take(node_edge_feat, dsts, axis=0)
    edge_feat = _edge_mlp(src_feat, dst_feat, dist_feat, w1, w2, w3, b_fc)
    lod = jnp.concatenate([node_feat, edge_feat], axis=0)

    a1 = jnp.zeros_like(lod).at[e2e_edge_index[1]].add(lod[e2e_edge_index[0]])
    ne = _mm_relu(a1, W_g)
    edge_feat2 = jnp.take(ne, eids, axis=0)

    lod2 = jnp.concatenate([node_feat, edge_feat2], axis=0)
    src2, dst2 = e2n_edge_index[0], e2n_edge_index[1]
    m = dst2 < N_NODES
    a2 = jnp.zeros((N_NODES + ROW_BLK, H), jnp.float32).at[
        jnp.where(m, dst2, N_NODES)].add(lod2[src2] * m[:, None])
    node_out = _mm_relu(a2[:N_NODES], W_s)

    return jnp.concatenate([node_out, edge_feat2], axis=0)


# speedup vs baseline: 1.6088x; 1.0681x over previous
import functools

import jax
import jax.numpy as jnp
from jax import lax
from jax.experimental import pallas as pl
from jax.experimental.pallas import tpu as pltpu
from jax.experimental.pallas import tpu_sc as plsc

N_NODES = 10000
N_EDGES = 320000
H = 128
ROW_BLK = 1000

_NC = 2
_NS = 16
_NW = _NC * _NS
_GC = 80


def _sc_mesh():
    return plsc.VectorSubcoreMesh(
        core_axis_name="c", subcore_axis_name="s",
        num_cores=_NC, num_subcores=_NS)


def _sc_gather(table, idx):
    b = idx.shape[0]
    per_w = b // _NW
    assert per_w * _NW == b and per_w % _GC == 0
    n_chunks = per_w // _GC

    @functools.partial(
        pl.kernel,
        out_type=jax.ShapeDtypeStruct((b, H), jnp.float32),
        mesh=_sc_mesh(),
        scratch_types=[
            pltpu.VMEM((2, _GC), jnp.int32),
            pltpu.VMEM((2, _GC, H), jnp.float32),
            pltpu.SemaphoreType.DMA,
            pltpu.SemaphoreType.DMA,
        ],
    )
    def gather_k(table_hbm, idx_hbm, out_hbm, idx_v, rows_v, sem0, sem1):
        wid = lax.axis_index("s") * _NC + lax.axis_index("c")
        base = wid * per_w
        sems = (sem0, sem1)

        def body(j, p):
            q = 1 - p

            @pl.when(j + 1 < n_chunks)
            def _():
                off = base + (j + 1) * _GC
                pltpu.sync_copy(idx_hbm.at[pl.ds(off, _GC)], idx_v.at[q])
                pltpu.async_copy(table_hbm.at[idx_v.at[q]], rows_v.at[q], sems[q])

            pltpu.make_async_copy(
                table_hbm.at[idx_v.at[p]], rows_v.at[p], sems[p]).wait()
            pltpu.sync_copy(rows_v.at[p], out_hbm.at[pl.ds(base + j * _GC, _GC)])

        pltpu.sync_copy(idx_hbm.at[pl.ds(base, _GC)], idx_v.at[0])
        pltpu.async_copy(table_hbm.at[idx_v.at[0]], rows_v.at[0], sem0)

        @pl.loop(0, 2 * (n_chunks // 2), step=2)
        def _(i):
            body(i, 0)
            body(i + 1, 1)

        if n_chunks % 2:
            body(n_chunks - 1, 0)

    return gather_k(table, idx)


def _mlp_body(src_ref, dst_ref, dist_ref, w1_ref, w2_ref, w3_ref, b_ref, o_ref):
    acc = jnp.dot(src_ref[...], w1_ref[...], preferred_element_type=jnp.float32)
    acc = acc + jnp.dot(dst_ref[...], w2_ref[...], preferred_element_type=jnp.float32)
    acc = acc + jnp.dot(dist_ref[...], w3_ref[...], preferred_element_type=jnp.float32)
    o_ref[...] = jnp.maximum(acc + b_ref[...], 0.0)


def _edge_mlp(sd_feat, dist_feat, w1, w2, w3, b):
    n = dist_feat.shape[0]
    grid = n // ROW_BLK
    row_spec = pl.BlockSpec((ROW_BLK, H), lambda i: (i, 0))
    dst_spec = pl.BlockSpec((ROW_BLK, H), lambda i: (i + n // ROW_BLK, 0))
    w_spec = pl.BlockSpec((H, H), lambda i: (0, 0))
    b_spec = pl.BlockSpec((1, H), lambda i: (0, 0))
    return pl.pallas_call(
        _mlp_body,
        grid=(grid,),
        in_specs=[row_spec, dst_spec, row_spec, w_spec, w_spec, w_spec, b_spec],
        out_specs=row_spec,
        out_shape=jax.ShapeDtypeStruct((n, H), jnp.float32),
    )(sd_feat, sd_feat, dist_feat, w1, w2, w3, b.reshape(1, H))


def _mm_relu_body(x_ref, w_ref, o_ref):
    o_ref[...] = jnp.maximum(
        jnp.dot(x_ref[...], w_ref[...], preferred_element_type=jnp.float32), 0.0)


def _mm_relu(x, w):
    n = x.shape[0]
    grid = n // ROW_BLK
    row_spec = pl.BlockSpec((ROW_BLK, H), lambda i: (i, 0))
    w_spec = pl.BlockSpec((H, H), lambda i: (0, 0))
    return pl.pallas_call(
        _mm_relu_body,
        grid=(grid,),
        in_specs=[row_spec, w_spec],
        out_specs=row_spec,
        out_shape=jax.ShapeDtypeStruct((n, H), jnp.float32),
    )(x, w)


def kernel(node_edge_feat, dist_feat_order, dist_feat, W_fc, b_fc, W_g, a_src_g,
           a_dst_g, W_s, W_e_s, a_src_s, a_dst_s, a_edge_s, srcs, dsts, nids,
           eids, e2n_edge_index, e2e_edge_index, nlod, elod):
    w1, w2, w3 = W_fc[:H], W_fc[H:2 * H], W_fc[2 * H:]
    node_feat = node_edge_feat[:N_NODES]

    sd_feat = _sc_gather(node_edge_feat, jnp.concatenate([srcs, dsts]))
    edge_feat = _edge_mlp(sd_feat, dist_feat, w1, w2, w3, b_fc)
    lod = jnp.concatenate([node_feat, edge_feat], axis=0)

    a1 = jnp.zeros_like(lod).at[e2e_edge_index[1]].add(lod[e2e_edge_index[0]])
    ne = _mm_relu(a1, W_g)
    edge_feat2 = _sc_gather(ne, eids)

    lod2 = jnp.concatenate([node_feat, edge_feat2], axis=0)
    src2, dst2 = e2n_edge_index[0], e2n_edge_index[1]
    m = dst2 < N_NODES
    a2 = jnp.zeros((N_NODES + ROW_BLK, H), jnp.float32).at[
        jnp.where(m, dst2, N_NODES)].add(lod2[src2] * m[:, None])
    node_out = _mm_relu(a2[:N_NODES], W_s)

    return jnp.concatenate([node_out, edge_feat2], axis=0)
